# Initial kernel scaffold; baseline (speedup 1.0000x reference)
import jax
import jax.numpy as jnp
from jax.experimental import pallas as pl


def kernel(weight1, root1, bias1, weight2, root2, bias2, lin_w, lin_b, edge_index, edge_type):
    raise NotImplementedError("write your pallas kernel here")


# speedup vs baseline: 10.9820x; 10.9820x over previous
import jax
import jax.numpy as jnp
from jax import lax
from jax.experimental import pallas as pl
from jax.experimental.pallas import tpu as pltpu
from jax.experimental.pallas import tpu_sc as plsc

N = 100000
R = 8
H = 32
HF = 16
C = 16
E = 1600000
NR = N * R

NC, NS = 2, 16
NW = NC * NS

KE = 2000
EPT = E // NS
EPW = E // NW
BPT = NR // NS

NP = 100096
SPT = NP // NS
KC = 1000
_ZR = 1000
_ZT = SPT - 6 * _ZR
_ZB = 10000

_MESH = plsc.VectorSubcoreMesh(
    core_axis_name="c", subcore_axis_name="s", num_cores=NC, num_subcores=NS
)


_EROWS = 2500
_ECOLS = 640


def _prep_body(src_ref, dst_ref, rel_ref, g1_ref, g2_ref, key_ref):
    s = src_ref[...]
    d = dst_ref[...]
    r = rel_ref[...]
    g1_ref[...] = r * N + s
    g2_ref[...] = s * R + r
    key_ref[...] = d * R + r


def _prep(src, dst, rel):
    grid = (_ECOLS // 128,)
    bs = pl.BlockSpec((_EROWS, 128), lambda i: (0, i))
    o = jax.ShapeDtypeStruct((_EROWS, _ECOLS), jnp.int32)
    g1, g2, key = pl.pallas_call(
        _prep_body,
        grid=grid,
        in_specs=[bs, bs, bs],
        out_specs=[bs, bs, bs],
        out_shape=[o, o, o],
    )(src.reshape(_EROWS, _ECOLS), dst.reshape(_EROWS, _ECOLS),
      rel.reshape(_EROWS, _ECOLS))
    return g1.reshape(E), g2.reshape(E), key.reshape(E)


_BW = 8000


def _split_body(w_ref, o_ref):
    x = w_ref[...]
    o_ref[0, :, :] = x[:, :HF]
    o_ref[1, :, :] = x[:, HF:]


def _split_w1(w1flat):
    grid = (NR // _BW,)
    return pl.pallas_call(
        _split_body,
        grid=grid,
        in_specs=[pl.BlockSpec((_BW, H), lambda i: (i, 0))],
        out_specs=pl.BlockSpec((NC, _BW, HF), lambda i: (0, i, 0)),
        out_shape=jax.ShapeDtypeStruct((NC, NR, HF), jnp.float32),
    )(w1flat)


def _hist_body(key_hbm, zeros_hbm, enorm_hbm, bins, key_v, ones_v, en_v, zb_v):
    cid = lax.axis_index("c")
    sid = lax.axis_index("s")

    pltpu.sync_copy(zeros_hbm.at[pl.ds(0, _ZB)], zb_v)

    def zfill(j, _):
        pltpu.sync_copy(zb_v, bins.at[pl.ds(sid * BPT + j * _ZB, _ZB)])
        return 0

    lax.fori_loop(0, BPT // _ZB, zfill, 0)

    def fill(j, _):
        ones_v[pl.ds(j * 16, 16)] = jnp.full((16,), 1.0, jnp.float32)
        return 0

    lax.fori_loop(0, KE // 16, fill, 0)
    plsc.subcore_barrier()

    def hchunk(g, _):
        off = pl.multiple_of(sid * EPT + g * KE, 8)
        pltpu.sync_copy(key_hbm.at[pl.ds(off, KE)], key_v)
        pltpu.sync_copy(ones_v, bins.at[key_v], add=True)
        return 0

    lax.fori_loop(0, EPT // KE, hchunk, 0)
    plsc.subcore_barrier()

    def echunk(g, _):
        off = pl.multiple_of((cid * NS + sid) * EPW + g * KE, 8)
        pltpu.sync_copy(key_hbm.at[pl.ds(off, KE)], key_v)
        pltpu.sync_copy(bins.at[key_v], en_v)

        def recip(j, _):
            sl = pl.ds(j * 16, 16)
            en_v[sl] = 1.0 / en_v[sl]
            return 0

        lax.fori_loop(0, KE // 16, recip, 0)
        pltpu.sync_copy(en_v, enorm_hbm.at[pl.ds(off, KE)])
        return 0

    lax.fori_loop(0, EPW // KE, echunk, 0)


def _hist_norm(key, zeros_nr):
    return pl.kernel(
        _hist_body,
        out_type=jax.ShapeDtypeStruct((E,), jnp.float32),
        mesh=_MESH,
        scratch_types=[
            pltpu.VMEM_SHARED((NR,), jnp.float32),
            pltpu.VMEM((KE,), jnp.int32),
            pltpu.VMEM((KE,), jnp.float32),
            pltpu.VMEM((KE,), jnp.float32),
            pltpu.VMEM((_ZB,), jnp.float32),
        ],
    )(key, zeros_nr)


def _conv_body(tbl_hbm, gidx_hbm, dst_hbm, en_hbm, zacc_hbm, agg_hbm,
               acc, gidx_v, dst_v, en_v, rows_v, sem):
    cid = lax.axis_index("c")
    sid = lax.axis_index("s")

    rs = sid * SPT
    pltpu.sync_copy(zacc_hbm, rows_v)

    def zfill(j, _):
        pltpu.sync_copy(rows_v, acc.at[pl.ds(rs + j * _ZR, _ZR), :])
        return 0

    lax.fori_loop(0, SPT // _ZR, zfill, 0)
    pltpu.sync_copy(rows_v.at[pl.ds(0, _ZT), :],
                    acc.at[pl.ds(rs + (SPT // _ZR) * _ZR, _ZT), :])
    plsc.subcore_barrier()

    tbl_c = tbl_hbm.at[cid]

    def chunk(g, _):
        off = pl.multiple_of(sid * EPT + g * KC, 8)
        pltpu.sync_copy(gidx_hbm.at[pl.ds(off, KC)], gidx_v)
        pltpu.sync_copy(dst_hbm.at[pl.ds(off, KC)], dst_v)
        pltpu.sync_copy(en_hbm.at[pl.ds(off, KC)], en_v)
        pltpu.async_copy(tbl_c.at[gidx_v], rows_v, sem).wait()

        @plsc.parallel_loop(0, KC, unroll=8)
        def _scale(i):
            e = plsc.load_gather(en_v, [jnp.full((16,), i, jnp.int32)])
            rows_v[i, :] = rows_v[i, :] * e

        pltpu.sync_copy(rows_v, acc.at[dst_v], add=True)
        return 0

    lax.fori_loop(0, EPT // KC, chunk, 0)
    plsc.subcore_barrier()

    def flush(j, _):
        pltpu.sync_copy(acc.at[pl.ds(rs + j * _ZR, _ZR), :], rows_v)
        pltpu.sync_copy(rows_v, agg_hbm.at[cid, pl.ds(rs + j * _ZR, _ZR), :])
        return 0

    lax.fori_loop(0, SPT // _ZR, flush, 0)
    tail = rs + (SPT // _ZR) * _ZR
    pltpu.sync_copy(acc.at[pl.ds(tail, _ZT), :], rows_v.at[pl.ds(0, _ZT), :])
    pltpu.sync_copy(rows_v.at[pl.ds(0, _ZT), :],
                    agg_hbm.at[cid, pl.ds(tail, _ZT), :])


def _conv_agg(tbl, gidx, dst, enorm, zacc):
    return pl.kernel(
        _conv_body,
        out_type=jax.ShapeDtypeStruct((NC, NP, HF), jnp.float32),
        mesh=_MESH,
        scratch_types=[
            pltpu.VMEM_SHARED((NP, HF), jnp.float32),
            pltpu.VMEM((KC,), jnp.int32),
            pltpu.VMEM((KC,), jnp.int32),
            pltpu.VMEM((KC,), jnp.float32),
            pltpu.VMEM((KC, HF), jnp.float32),
            pltpu.SemaphoreType.DMA,
        ],
        compiler_params=pltpu.CompilerParams(
            use_tc_tiling_on_sc=False, needs_layout_passes=False
        ),
    )(tbl, gidx, dst, enorm, zacc)


_BN = 1000


def _dense1_body(agg_ref, root1_ref, b1_ref, w2_ref, root2_ref, b2_ref,
                 xw_ref, z_ref):
    a = agg_ref[...]
    h = jnp.concatenate([a[0], a[1]], axis=1) + root1_ref[...] + b1_ref[...]
    h = jnp.maximum(h, 0.0)
    w2 = w2_ref[...]
    outs = [jnp.dot(h, w2[r], preferred_element_type=jnp.float32)
            for r in range(R)]
    hw = jnp.stack(outs, axis=1)
    xw_ref[0, :, :] = hw[:, :, :HF].reshape(_BN * R, HF)
    xw_ref[1, :, :] = hw[:, :, HF:].reshape(_BN * R, HF)
    z_ref[...] = (jnp.dot(h, root2_ref[...], preferred_element_type=jnp.float32)
                  + b2_ref[...])


def _dense1(agg1, root1, bias1, w2, root2, bias2):
    grid = (N // _BN,)
    xw, z = pl.pallas_call(
        _dense1_body,
        grid=grid,
        in_specs=[
            pl.BlockSpec((NC, _BN, HF), lambda i: (0, i, 0)),
            pl.BlockSpec((_BN, H), lambda i: (i, 0)),
            pl.BlockSpec((1, H), lambda i: (0, 0)),
            pl.BlockSpec((R, H, H), lambda i: (0, 0, 0)),
            pl.BlockSpec((H, H), lambda i: (0, 0)),
            pl.BlockSpec((1, H), lambda i: (0, 0)),
        ],
        out_specs=[
            pl.BlockSpec((NC, _BN * R, HF), lambda i: (0, i, 0)),
            pl.BlockSpec((_BN, H), lambda i: (i, 0)),
        ],
        out_shape=[
            jax.ShapeDtypeStruct((NC, N * R, HF), jnp.float32),
            jax.ShapeDtypeStruct((N, H), jnp.float32),
        ],
    )(agg1, root1, bias1.reshape(1, H), w2, root2, bias2.reshape(1, H))
    return xw, z


def _final_body(agg_ref, z_ref, lw_ref, lb_ref, o_ref):
    a = agg_ref[...]
    h2 = jnp.concatenate([a[0], a[1]], axis=1) + z_ref[...]
    h2 = jnp.maximum(h2, 0.0)
    o_ref[...] = (jnp.dot(h2, lw_ref[...], preferred_element_type=jnp.float32)
                  + lb_ref[...])


def _final(agg2, z, lin_w, lin_b):
    grid = (N // _BN,)
    return pl.pallas_call(
        _final_body,
        grid=grid,
        in_specs=[
            pl.BlockSpec((NC, _BN, HF), lambda i: (0, i, 0)),
            pl.BlockSpec((_BN, H), lambda i: (i, 0)),
            pl.BlockSpec((H, C), lambda i: (0, 0)),
            pl.BlockSpec((1, C), lambda i: (0, 0)),
        ],
        out_specs=pl.BlockSpec((_BN, C), lambda i: (i, 0)),
        out_shape=jax.ShapeDtypeStruct((N, C), jnp.float32),
    )(agg2, z, lin_w, lin_b.reshape(1, C))


def kernel(weight1, root1, bias1, weight2, root2, bias2, lin_w, lin_b,
           edge_index, edge_type):
    src = edge_index[0]
    dst = edge_index[1]
    gidx1, gidx2, key = _prep(src, dst, edge_type)

    zeros_nr = jnp.zeros((NR,), jnp.float32)
    zacc = jnp.zeros((_ZR, HF), jnp.float32)

    enorm = _hist_norm(key, zeros_nr)

    w1s = _split_w1(weight1.reshape(NR, H))
    agg1 = _conv_agg(w1s, gidx1, dst, enorm, zacc)
    xw, z = _dense1(agg1, root1, bias1, weight2, root2, bias2)
    agg2 = _conv_agg(xw, gidx2, dst, enorm, zacc)
    return _final(agg2, z, lin_w, lin_b)


# speedup vs baseline: 18.1042x; 1.6485x over previous
import jax
import jax.numpy as jnp
from jax import lax
from jax.experimental import pallas as pl
from jax.experimental.pallas import tpu as pltpu
from jax.experimental.pallas import tpu_sc as plsc

N = 100000
R = 8
H = 32
HF = 16
C = 16
E = 1600000
NR = N * R

NC, NS = 2, 16
NW = NC * NS

KE = 2000
EPT = E // NS
EPW = E // NW
BPT = NR // NS
_ZB = 10000

NP = 100096
SPT = NP // NS
KC = 800
_ZR = 800
_NZ = SPT // _ZR
_ZT = SPT - _NZ * _ZR

_MESH = plsc.VectorSubcoreMesh(
    core_axis_name="c", subcore_axis_name="s", num_cores=NC, num_subcores=NS
)


_EROWS = 2500
_ECOLS = 640


def _prep_body(src_ref, dst_ref, rel_ref, g1_ref, g2_ref, key_ref):
    s = src_ref[...]
    d = dst_ref[...]
    r = rel_ref[...]
    t1 = (r * N + s) * 2
    g1_ref[0, :, :] = t1
    g1_ref[1, :, :] = t1 + 1
    t2 = s * R + r
    g2_ref[0, :, :] = t2
    g2_ref[1, :, :] = t2 + N * R
    key_ref[...] = d * R + r


def _prep(src, dst, rel):
    grid = (_ECOLS // 128,)
    bs = pl.BlockSpec((_EROWS, 128), lambda i: (0, i))
    bs2 = pl.BlockSpec((NC, _EROWS, 128), lambda i: (0, 0, i))
    o = jax.ShapeDtypeStruct((_EROWS, _ECOLS), jnp.int32)
    o2 = jax.ShapeDtypeStruct((NC, _EROWS, _ECOLS), jnp.int32)
    g1, g2, key = pl.pallas_call(
        _prep_body,
        grid=grid,
        in_specs=[bs, bs, bs],
        out_specs=[bs2, bs2, bs],
        out_shape=[o2, o2, o],
    )(src.reshape(_EROWS, _ECOLS), dst.reshape(_EROWS, _ECOLS),
      rel.reshape(_EROWS, _ECOLS))
    return g1.reshape(NC, E), g2.reshape(NC, E), key.reshape(E)


def _hist_body(key_hbm, enorm_hbm, bins, key_v, ones_v, en_v, zb_v):
    cid = lax.axis_index("c")
    sid = lax.axis_index("s")

    z16 = jnp.zeros((16,), jnp.float32)

    def zv(j, _):
        zb_v[pl.ds(j * 16, 16)] = z16
        return 0

    lax.fori_loop(0, _ZB // 16, zv, 0)

    def zfill(j, _):
        pltpu.sync_copy(zb_v, bins.at[pl.ds(sid * BPT + j * _ZB, _ZB)])
        return 0

    lax.fori_loop(0, BPT // _ZB, zfill, 0)

    o16 = jnp.full((16,), 1.0, jnp.float32)

    def fill(j, _):
        ones_v[pl.ds(j * 16, 16)] = o16
        return 0

    lax.fori_loop(0, KE // 16, fill, 0)
    plsc.subcore_barrier()

    def hchunk(g, _):
        off = pl.multiple_of(sid * EPT + g * KE, 8)
        pltpu.sync_copy(key_hbm.at[pl.ds(off, KE)], key_v)
        pltpu.sync_copy(ones_v, bins.at[key_v], add=True)
        return 0

    lax.fori_loop(0, EPT // KE, hchunk, 0)
    plsc.subcore_barrier()

    def echunk(g, _):
        off = pl.multiple_of((cid * NS + sid) * EPW + g * KE, 8)
        pltpu.sync_copy(key_hbm.at[pl.ds(off, KE)], key_v)
        pltpu.sync_copy(bins.at[key_v], en_v)

        def recip(j, _):
            sl = pl.ds(j * 16, 16)
            en_v[sl] = 1.0 / en_v[sl]
            return 0

        lax.fori_loop(0, KE // 16, recip, 0)
        pltpu.sync_copy(en_v, enorm_hbm.at[pl.ds(off, KE)])
        return 0

    lax.fori_loop(0, EPW // KE, echunk, 0)


def _hist_norm(key):
    return pl.kernel(
        _hist_body,
        out_type=jax.ShapeDtypeStruct((E,), jnp.float32),
        mesh=_MESH,
        scratch_types=[
            pltpu.VMEM_SHARED((NR,), jnp.float32),
            pltpu.VMEM((KE,), jnp.int32),
            pltpu.VMEM((KE,), jnp.float32),
            pltpu.VMEM((KE,), jnp.float32),
            pltpu.VMEM((_ZB,), jnp.float32),
        ],
    )(key)


def _conv_body(tbl_hbm, gidx_hbm, dst_hbm, en_hbm, agg_hbm,
               acc, gidx_v, dst_v, en_v, rows_v, sem):
    cid = lax.axis_index("c")
    sid = lax.axis_index("s")

    z16 = jnp.zeros((16,), jnp.float32)

    @plsc.parallel_loop(0, KC, unroll=8)
    def _zrows(i):
        rows_v[i, :] = z16

    rs = sid * SPT

    def zfill(j, _):
        pltpu.sync_copy(rows_v, acc.at[pl.ds(rs + j * _ZR, _ZR), :])
        return 0

    lax.fori_loop(0, _NZ, zfill, 0)
    pltpu.sync_copy(rows_v.at[pl.ds(0, _ZT), :],
                    acc.at[pl.ds(rs + _NZ * _ZR, _ZT), :])
    plsc.subcore_barrier()

    gc = gidx_hbm.at[cid]

    def chunk(g, _):
        off = pl.multiple_of(sid * EPT + g * KC, 8)
        pltpu.sync_copy(gc.at[pl.ds(off, KC)], gidx_v)
        pltpu.sync_copy(dst_hbm.at[pl.ds(off, KC)], dst_v)
        pltpu.sync_copy(en_hbm.at[pl.ds(off, KC)], en_v)
        pltpu.async_copy(tbl_hbm.at[gidx_v], rows_v, sem).wait()

        @plsc.parallel_loop(0, KC, unroll=8)
        def _scale(i):
            e = plsc.load_gather(en_v, [jnp.full((16,), i, jnp.int32)])
            rows_v[i, :] = rows_v[i, :] * e

        pltpu.sync_copy(rows_v, acc.at[dst_v], add=True)
        return 0

    lax.fori_loop(0, EPT // KC, chunk, 0)
    plsc.subcore_barrier()

    def flush(j, _):
        pltpu.sync_copy(acc.at[pl.ds(rs + j * _ZR, _ZR), :], rows_v)
        pltpu.sync_copy(rows_v, agg_hbm.at[cid, pl.ds(rs + j * _ZR, _ZR), :])
        return 0

    lax.fori_loop(0, _NZ, flush, 0)
    tail = rs + _NZ * _ZR
    pltpu.sync_copy(acc.at[pl.ds(tail, _ZT), :], rows_v.at[pl.ds(0, _ZT), :])
    pltpu.sync_copy(rows_v.at[pl.ds(0, _ZT), :],
                    agg_hbm.at[cid, pl.ds(tail, _ZT), :])


def _conv_agg(tbl, gidx, dst, enorm):
    return pl.kernel(
        _conv_body,
        out_type=jax.ShapeDtypeStruct((NC, NP, HF), jnp.float32),
        mesh=_MESH,
        scratch_types=[
            pltpu.VMEM_SHARED((NP, HF), jnp.float32),
            pltpu.VMEM((KC,), jnp.int32),
            pltpu.VMEM((KC,), jnp.int32),
            pltpu.VMEM((KC,), jnp.float32),
            pltpu.VMEM((KC, HF), jnp.float32),
            pltpu.SemaphoreType.DMA,
        ],
        compiler_params=pltpu.CompilerParams(
            use_tc_tiling_on_sc=False, needs_layout_passes=False
        ),
    )(tbl, gidx, dst, enorm)


def _w2cat_body(w2_ref, o_ref):
    w2 = w2_ref[...]
    for c in range(NC):
        o_ref[c, :, :] = jnp.concatenate(
            [w2[r][:, c * HF:(c + 1) * HF] for r in range(R)], axis=1)


def _w2cat(w2):
    return pl.pallas_call(
        _w2cat_body,
        out_shape=jax.ShapeDtypeStruct((NC, H, R * HF), jnp.float32),
    )(w2)


_BN = 1000


def _dense1_body(agg_ref, root1_ref, b1_ref, w2c_ref, root2_ref, b2_ref,
                 xw_ref, z_ref):
    a = agg_ref[...]
    h = jnp.concatenate([a[0], a[1]], axis=1) + root1_ref[...] + b1_ref[...]
    h = jnp.maximum(h, 0.0)
    w2c = w2c_ref[...]
    for c in range(NC):
        xw_ref[c, :, :] = jnp.dot(h, w2c[c],
                                  preferred_element_type=jnp.float32)
    z_ref[...] = (jnp.dot(h, root2_ref[...], preferred_element_type=jnp.float32)
                  + b2_ref[...])


def _dense1(agg1, root1, bias1, w2c, root2, bias2):
    grid = (N // _BN,)
    xw, z = pl.pallas_call(
        _dense1_body,
        grid=grid,
        in_specs=[
            pl.BlockSpec((NC, _BN, HF), lambda i: (0, i, 0)),
            pl.BlockSpec((_BN, H), lambda i: (i, 0)),
            pl.BlockSpec((1, H), lambda i: (0, 0)),
            pl.BlockSpec((NC, H, R * HF), lambda i: (0, 0, 0)),
            pl.BlockSpec((H, H), lambda i: (0, 0)),
            pl.BlockSpec((1, H), lambda i: (0, 0)),
        ],
        out_specs=[
            pl.BlockSpec((NC, _BN, R * HF), lambda i: (0, i, 0)),
            pl.BlockSpec((_BN, H), lambda i: (i, 0)),
        ],
        out_shape=[
            jax.ShapeDtypeStruct((NC, N, R * HF), jnp.float32),
            jax.ShapeDtypeStruct((N, H), jnp.float32),
        ],
    )(agg1, root1, bias1.reshape(1, H), w2c, root2, bias2.reshape(1, H))
    return xw, z


def _final_body(agg_ref, z_ref, lw_ref, lb_ref, o_ref):
    a = agg_ref[...]
    h2 = jnp.concatenate([a[0], a[1]], axis=1) + z_ref[...]
    h2 = jnp.maximum(h2, 0.0)
    o_ref[...] = (jnp.dot(h2, lw_ref[...], preferred_element_type=jnp.float32)
                  + lb_ref[...])


def _final(agg2, z, lin_w, lin_b):
    grid = (N // _BN,)
    return pl.pallas_call(
        _final_body,
        grid=grid,
        in_specs=[
            pl.BlockSpec((NC, _BN, HF), lambda i: (0, i, 0)),
            pl.BlockSpec((_BN, H), lambda i: (i, 0)),
            pl.BlockSpec((H, C), lambda i: (0, 0)),
            pl.BlockSpec((1, C), lambda i: (0, 0)),
        ],
        out_specs=pl.BlockSpec((_BN, C), lambda i: (i, 0)),
        out_shape=jax.ShapeDtypeStruct((N, C), jnp.float32),
    )(agg2, z, lin_w, lin_b.reshape(1, C))


def kernel(weight1, root1, bias1, weight2, root2, bias2, lin_w, lin_b,
           edge_index, edge_type):
    src = edge_index[0]
    dst = edge_index[1]
    gidx1, gidx2, key = _prep(src, dst, edge_type)

    enorm = _hist_norm(key)

    w1rows = weight1.reshape(NR * NC, HF)
    agg1 = _conv_agg(w1rows, gidx1, dst, enorm)

    w2c = _w2cat(weight2)
    xw, z = _dense1(agg1, root1, bias1, w2c, root2, bias2)

    xwrows = xw.reshape(NC * N * R, HF)
    agg2 = _conv_agg(xwrows, gidx2, dst, enorm)
    return _final(agg2, z, lin_w, lin_b)


# speedup vs baseline: 22.1957x; 1.2260x over previous
import jax
import jax.numpy as jnp
from jax import lax
from jax.experimental import pallas as pl
from jax.experimental.pallas import tpu as pltpu
from jax.experimental.pallas import tpu_sc as plsc

N = 100000
R = 8
H = 32
HF = 16
C = 16
E = 1600000
NR = N * R

NC, NS = 2, 16
NW = NC * NS

KE = 2000
EPT = E // NS
EPW = E // NW
BPT = NR // NS
_ZB = 10000

NP = 100096
SPT = NP // NS
KC = 800
NCH = EPT // KC
NPAIR = (NCH - 1) // 2
_ZR = 800
_NZ = SPT // _ZR
_ZT = SPT - _NZ * _ZR

_MESH = plsc.VectorSubcoreMesh(
    core_axis_name="c", subcore_axis_name="s", num_cores=NC, num_subcores=NS
)


_EROWS = 2500
_ECOLS = 640


def _prep_body(src_ref, dst_ref, rel_ref, g1_ref, g2_ref, key_ref):
    s = src_ref[...]
    d = dst_ref[...]
    r = rel_ref[...]
    t1 = (r * N + s) * 2
    g1_ref[0, :, :] = t1
    g1_ref[1, :, :] = t1 + 1
    t2 = s * R + r
    g2_ref[0, :, :] = t2
    g2_ref[1, :, :] = t2 + N * R
    key_ref[...] = d * R + r


def _prep(src, dst, rel):
    grid = (_ECOLS // 128,)
    bs = pl.BlockSpec((_EROWS, 128), lambda i: (0, i))
    bs2 = pl.BlockSpec((NC, _EROWS, 128), lambda i: (0, 0, i))
    o = jax.ShapeDtypeStruct((_EROWS, _ECOLS), jnp.int32)
    o2 = jax.ShapeDtypeStruct((NC, _EROWS, _ECOLS), jnp.int32)
    g1, g2, key = pl.pallas_call(
        _prep_body,
        grid=grid,
        in_specs=[bs, bs, bs],
        out_specs=[bs2, bs2, bs],
        out_shape=[o2, o2, o],
    )(src.reshape(_EROWS, _ECOLS), dst.reshape(_EROWS, _ECOLS),
      rel.reshape(_EROWS, _ECOLS))
    return g1.reshape(NC, E), g2.reshape(NC, E), key.reshape(E)


def _hist_body(key_hbm, enorm_hbm, bins, key_v0, key_v1, ones_v, en_v, zb_v,
               ksem0, ksem1):
    cid = lax.axis_index("c")
    sid = lax.axis_index("s")

    z16 = jnp.zeros((16,), jnp.float32)

    def zv(j, _):
        zb_v[pl.ds(j * 16, 16)] = z16
        return 0

    lax.fori_loop(0, _ZB // 16, zv, 0)

    def zfill(j, _):
        pltpu.sync_copy(zb_v, bins.at[pl.ds(sid * BPT + j * _ZB, _ZB)])
        return 0

    lax.fori_loop(0, BPT // _ZB, zfill, 0)

    o16 = jnp.full((16,), 1.0, jnp.float32)

    def fill(j, _):
        ones_v[pl.ds(j * 16, 16)] = o16
        return 0

    lax.fori_loop(0, KE // 16, fill, 0)
    plsc.subcore_barrier()

    nh = EPT // KE
    kbufs = (key_v0, key_v1)
    ksems = (ksem0, ksem1)

    def _kargs(g, b):
        off = pl.multiple_of(sid * EPT + g * KE, 8)
        return key_hbm.at[pl.ds(off, KE)], kbufs[b], ksems[b]

    pltpu.async_copy(*_kargs(0, 0))
    pltpu.async_copy(*_kargs(1, 1))

    def hchunk2(j, _):
        for b in range(2):
            g = 2 * j + b
            pltpu.make_async_copy(*_kargs(g, b)).wait()
            pltpu.sync_copy(ones_v, bins.at[kbufs[b]], add=True)

            @pl.when(g + 2 < nh)
            def _():
                pltpu.async_copy(*_kargs(g + 2, b))
        return 0

    lax.fori_loop(0, nh // 2, hchunk2, 0)
    plsc.subcore_barrier()

    def echunk(g, _):
        off = pl.multiple_of((cid * NS + sid) * EPW + g * KE, 8)
        pltpu.sync_copy(key_hbm.at[pl.ds(off, KE)], key_v0)
        pltpu.sync_copy(bins.at[key_v0], en_v)

        def recip(j, _):
            sl = pl.ds(j * 16, 16)
            en_v[sl] = 1.0 / en_v[sl]
            return 0

        lax.fori_loop(0, KE // 16, recip, 0)
        pltpu.sync_copy(en_v, enorm_hbm.at[pl.ds(off, KE)])
        return 0

    lax.fori_loop(0, EPW // KE, echunk, 0)


def _hist_norm(key):
    return pl.kernel(
        _hist_body,
        out_type=jax.ShapeDtypeStruct((E,), jnp.float32),
        mesh=_MESH,
        scratch_types=[
            pltpu.VMEM_SHARED((NR,), jnp.float32),
            pltpu.VMEM((KE,), jnp.int32),
            pltpu.VMEM((KE,), jnp.int32),
            pltpu.VMEM((KE,), jnp.float32),
            pltpu.VMEM((KE,), jnp.float32),
            pltpu.VMEM((_ZB,), jnp.float32),
            pltpu.SemaphoreType.DMA,
            pltpu.SemaphoreType.DMA,
        ],
    )(key)


def _conv_body(tbl_hbm, gidx_hbm, dst_hbm, en_hbm, agg_hbm,
               acc, gidx_v0, gidx_v1, dst_v0, dst_v1, en_v0, en_v1,
               rows_v0, rows_v1, sem0, sem1):
    cid = lax.axis_index("c")
    sid = lax.axis_index("s")

    z16 = jnp.zeros((16,), jnp.float32)
    r0 = rows_v0

    @plsc.parallel_loop(0, KC, unroll=8)
    def _zrows(i):
        rows_v0[i, :] = z16

    rs = sid * SPT

    def zfill(j, _):
        pltpu.sync_copy(r0, acc.at[pl.ds(rs + j * _ZR, _ZR), :])
        return 0

    lax.fori_loop(0, _NZ, zfill, 0)
    pltpu.sync_copy(r0.at[pl.ds(0, _ZT), :],
                    acc.at[pl.ds(rs + _NZ * _ZR, _ZT), :])
    plsc.subcore_barrier()

    gc = gidx_hbm.at[cid]
    gbufs = (gidx_v0, gidx_v1)
    dbufs = (dst_v0, dst_v1)
    ebufs = (en_v0, en_v1)
    rbufs = (rows_v0, rows_v1)
    sems = (sem0, sem1)

    def _issue(g, b):
        off = pl.multiple_of(sid * EPT + g * KC, 8)
        pltpu.sync_copy(gc.at[pl.ds(off, KC)], gbufs[b])
        pltpu.sync_copy(dst_hbm.at[pl.ds(off, KC)], dbufs[b])
        pltpu.sync_copy(en_hbm.at[pl.ds(off, KC)], ebufs[b])
        pltpu.async_copy(tbl_hbm.at[gbufs[b]], rbufs[b], sems[b])

    _issue(0, 0)
    _issue(1, 1)

    def _drain(b):
        pltpu.make_async_copy(tbl_hbm.at[gbufs[b]], rbufs[b],
                              sems[b]).wait()
        rb = rbufs[b]
        eb = ebufs[b]

        @plsc.parallel_loop(0, KC, unroll=8)
        def _scale(i):
            e = plsc.load_gather(eb, [jnp.full((16,), i, jnp.int32)])
            rb[i, :] = rb[i, :] * e

        pltpu.sync_copy(rb, acc.at[dbufs[b]], add=True)

    def chunk2(j, _):
        for b in range(2):
            g = 2 * j + b
            _drain(b)

            @pl.when(g + 2 < NCH)
            def _():
                _issue(g + 2, b)
        return 0

    lax.fori_loop(0, NPAIR, chunk2, 0)
    _drain(0)
    plsc.subcore_barrier()

    def flush(j, _):
        pltpu.sync_copy(acc.at[pl.ds(rs + j * _ZR, _ZR), :], r0)
        pltpu.sync_copy(r0, agg_hbm.at[cid, pl.ds(rs + j * _ZR, _ZR), :])
        return 0

    lax.fori_loop(0, _NZ, flush, 0)
    tail = rs + _NZ * _ZR
    pltpu.sync_copy(acc.at[pl.ds(tail, _ZT), :], r0.at[pl.ds(0, _ZT), :])
    pltpu.sync_copy(r0.at[pl.ds(0, _ZT), :],
                    agg_hbm.at[cid, pl.ds(tail, _ZT), :])


def _conv_agg(tbl, gidx, dst, enorm):
    return pl.kernel(
        _conv_body,
        out_type=jax.ShapeDtypeStruct((NC, NP, HF), jnp.float32),
        mesh=_MESH,
        scratch_types=[
            pltpu.VMEM_SHARED((NP, HF), jnp.float32),
            pltpu.VMEM((KC,), jnp.int32),
            pltpu.VMEM((KC,), jnp.int32),
            pltpu.VMEM((KC,), jnp.int32),
            pltpu.VMEM((KC,), jnp.int32),
            pltpu.VMEM((KC,), jnp.float32),
            pltpu.VMEM((KC,), jnp.float32),
            pltpu.VMEM((KC, HF), jnp.float32),
            pltpu.VMEM((KC, HF), jnp.float32),
            pltpu.SemaphoreType.DMA,
            pltpu.SemaphoreType.DMA,
        ],
        compiler_params=pltpu.CompilerParams(
            use_tc_tiling_on_sc=False, needs_layout_passes=False
        ),
    )(tbl, gidx, dst, enorm)


def _w2cat_body(w2_ref, o_ref):
    w2 = w2_ref[...]
    for c in range(NC):
        o_ref[c, :, :] = jnp.concatenate(
            [w2[r][:, c * HF:(c + 1) * HF] for r in range(R)], axis=1)


def _w2cat(w2):
    return pl.pallas_call(
        _w2cat_body,
        out_shape=jax.ShapeDtypeStruct((NC, H, R * HF), jnp.float32),
    )(w2)


_BN = 1000


def _dense1_body(agg_ref, root1_ref, b1_ref, w2c_ref, root2_ref, b2_ref,
                 xw_ref, z_ref):
    a = agg_ref[...]
    h = jnp.concatenate([a[0], a[1]], axis=1) + root1_ref[...] + b1_ref[...]
    h = jnp.maximum(h, 0.0)
    w2c = w2c_ref[...]
    for c in range(NC):
        xw_ref[c, :, :] = jnp.dot(h, w2c[c],
                                  preferred_element_type=jnp.float32)
    z_ref[...] = (jnp.dot(h, root2_ref[...], preferred_element_type=jnp.float32)
                  + b2_ref[...])


def _dense1(agg1, root1, bias1, w2c, root2, bias2):
    grid = (N // _BN,)
    xw, z = pl.pallas_call(
        _dense1_body,
        grid=grid,
        in_specs=[
            pl.BlockSpec((NC, _BN, HF), lambda i: (0, i, 0)),
            pl.BlockSpec((_BN, H), lambda i: (i, 0)),
            pl.BlockSpec((1, H), lambda i: (0, 0)),
            pl.BlockSpec((NC, H, R * HF), lambda i: (0, 0, 0)),
            pl.BlockSpec((H, H), lambda i: (0, 0)),
            pl.BlockSpec((1, H), lambda i: (0, 0)),
        ],
        out_specs=[
            pl.BlockSpec((NC, _BN, R * HF), lambda i: (0, i, 0)),
            pl.BlockSpec((_BN, H), lambda i: (i, 0)),
        ],
        out_shape=[
            jax.ShapeDtypeStruct((NC, N, R * HF), jnp.float32),
            jax.ShapeDtypeStruct((N, H), jnp.float32),
        ],
    )(agg1, root1, bias1.reshape(1, H), w2c, root2, bias2.reshape(1, H))
    return xw, z


def _final_body(agg_ref, z_ref, lw_ref, lb_ref, o_ref):
    a = agg_ref[...]
    h2 = jnp.concatenate([a[0], a[1]], axis=1) + z_ref[...]
    h2 = jnp.maximum(h2, 0.0)
    o_ref[...] = (jnp.dot(h2, lw_ref[...], preferred_element_type=jnp.float32)
                  + lb_ref[...])


def _final(agg2, z, lin_w, lin_b):
    grid = (N // _BN,)
    return pl.pallas_call(
        _final_body,
        grid=grid,
        in_specs=[
            pl.BlockSpec((NC, _BN, HF), lambda i: (0, i, 0)),
            pl.BlockSpec((_BN, H), lambda i: (i, 0)),
            pl.BlockSpec((H, C), lambda i: (0, 0)),
            pl.BlockSpec((1, C), lambda i: (0, 0)),
        ],
        out_specs=pl.BlockSpec((_BN, C), lambda i: (i, 0)),
        out_shape=jax.ShapeDtypeStruct((N, C), jnp.float32),
    )(agg2, z, lin_w, lin_b.reshape(1, C))


def kernel(weight1, root1, bias1, weight2, root2, bias2, lin_w, lin_b,
           edge_index, edge_type):
    src = edge_index[0]
    dst = edge_index[1]
    gidx1, gidx2, key = _prep(src, dst, edge_type)

    enorm = _hist_norm(key)

    w1rows = weight1.reshape(NR * NC, HF)
    agg1 = _conv_agg(w1rows, gidx1, dst, enorm)

    w2c = _w2cat(weight2)
    xw, z = _dense1(agg1, root1, bias1, w2c, root2, bias2)

    xwrows = xw.reshape(NC * N * R, HF)
    agg2 = _conv_agg(xwrows, gidx2, dst, enorm)
    return _final(agg2, z, lin_w, lin_b)


# speedup vs baseline: 22.5532x; 1.0161x over previous
import jax
import jax.numpy as jnp
from jax import lax
from jax.experimental import pallas as pl
from jax.experimental.pallas import tpu as pltpu
from jax.experimental.pallas import tpu_sc as plsc

N = 100000
R = 8
H = 32
HF = 16
C = 16
E = 1600000
NR = N * R

NC, NS = 2, 16
NW = NC * NS

KE = 2000
EPT = E // NS
EPW = E // NW
BPT = NR // NS
_ZB = 10000

NP = 100096
SPT = NP // NS
KC = 800
NCH = EPT // KC
NPAIR = (NCH - 1) // 2
_ZR = 800
_NZ = SPT // _ZR
_ZT = SPT - _NZ * _ZR

_MESH = plsc.VectorSubcoreMesh(
    core_axis_name="c", subcore_axis_name="s", num_cores=NC, num_subcores=NS
)


_EROWS = 2500
_ECOLS = 640


def _prep_body(src_ref, dst_ref, rel_ref, g1_ref, g2_ref, key_ref):
    s = src_ref[...]
    d = dst_ref[...]
    r = rel_ref[...]
    t1 = (r * N + s) * 2
    g1_ref[0, :, :] = t1
    g1_ref[1, :, :] = t1 + 1
    t2 = s * R + r
    g2_ref[0, :, :] = t2
    g2_ref[1, :, :] = t2 + N * R
    key_ref[...] = d * R + r


def _prep(src, dst, rel):
    grid = (_ECOLS // 128,)
    bs = pl.BlockSpec((_EROWS, 128), lambda i: (0, i))
    bs2 = pl.BlockSpec((NC, _EROWS, 128), lambda i: (0, 0, i))
    o = jax.ShapeDtypeStruct((_EROWS, _ECOLS), jnp.int32)
    o2 = jax.ShapeDtypeStruct((NC, _EROWS, _ECOLS), jnp.int32)
    g1, g2, key = pl.pallas_call(
        _prep_body,
        grid=grid,
        in_specs=[bs, bs, bs],
        out_specs=[bs2, bs2, bs],
        out_shape=[o2, o2, o],
    )(src.reshape(_EROWS, _ECOLS), dst.reshape(_EROWS, _ECOLS),
      rel.reshape(_EROWS, _ECOLS))
    return g1.reshape(NC, E), g2.reshape(NC, E), key.reshape(E)


def _hist_body(key_hbm, enorm_hbm, bins, key_v0, key_v1, ones_v, en_v, zb_v,
               ksem0, ksem1):
    cid = lax.axis_index("c")
    sid = lax.axis_index("s")

    z16 = jnp.zeros((16,), jnp.float32)

    def zv(j, _):
        zb_v[pl.ds(j * 16, 16)] = z16
        return 0

    lax.fori_loop(0, _ZB // 16, zv, 0)

    def zfill(j, _):
        pltpu.sync_copy(zb_v, bins.at[pl.ds(sid * BPT + j * _ZB, _ZB)])
        return 0

    lax.fori_loop(0, BPT // _ZB, zfill, 0)

    o16 = jnp.full((16,), 1.0, jnp.float32)

    def fill(j, _):
        ones_v[pl.ds(j * 16, 16)] = o16
        return 0

    lax.fori_loop(0, KE // 16, fill, 0)
    plsc.subcore_barrier()

    nh = EPT // KE
    kbufs = (key_v0, key_v1)
    ksems = (ksem0, ksem1)

    def _kargs(g, b):
        off = pl.multiple_of(sid * EPT + g * KE, 8)
        return key_hbm.at[pl.ds(off, KE)], kbufs[b], ksems[b]

    pltpu.async_copy(*_kargs(0, 0))
    pltpu.async_copy(*_kargs(1, 1))

    def hchunk2(j, _):
        for b in range(2):
            g = 2 * j + b
            pltpu.make_async_copy(*_kargs(g, b)).wait()
            pltpu.sync_copy(ones_v, bins.at[kbufs[b]], add=True)

            @pl.when(g + 2 < nh)
            def _():
                pltpu.async_copy(*_kargs(g + 2, b))
        return 0

    lax.fori_loop(0, nh // 2, hchunk2, 0)
    plsc.subcore_barrier()

    def echunk(g, _):
        off = pl.multiple_of((cid * NS + sid) * EPW + g * KE, 8)
        pltpu.sync_copy(key_hbm.at[pl.ds(off, KE)], key_v0)
        pltpu.sync_copy(bins.at[key_v0], en_v)

        def recip(j, _):
            sl = pl.ds(j * 16, 16)
            en_v[sl] = 1.0 / en_v[sl]
            return 0

        lax.fori_loop(0, KE // 16, recip, 0)
        pltpu.sync_copy(en_v, enorm_hbm.at[pl.ds(off, KE)])
        return 0

    lax.fori_loop(0, EPW // KE, echunk, 0)


def _hist_norm(key):
    return pl.kernel(
        _hist_body,
        out_type=jax.ShapeDtypeStruct((E,), jnp.float32),
        mesh=_MESH,
        scratch_types=[
            pltpu.VMEM_SHARED((NR,), jnp.float32),
            pltpu.VMEM((KE,), jnp.int32),
            pltpu.VMEM((KE,), jnp.int32),
            pltpu.VMEM((KE,), jnp.float32),
            pltpu.VMEM((KE,), jnp.float32),
            pltpu.VMEM((_ZB,), jnp.float32),
            pltpu.SemaphoreType.DMA,
            pltpu.SemaphoreType.DMA,
        ],
    )(key)


def _conv_body(tbl_hbm, gidx_hbm, dst_hbm, en_hbm, agg_hbm,
               acc, gidx_v0, gidx_v1, dst_v0, dst_v1, en_v0, en_v1,
               rows_v0, rows_v1, sem0, sem1):
    cid = lax.axis_index("c")
    sid = lax.axis_index("s")

    z16 = jnp.zeros((16,), jnp.float32)
    r0 = rows_v0

    @plsc.parallel_loop(0, KC, unroll=8)
    def _zrows(i):
        rows_v0[i, :] = z16

    rs = sid * SPT

    def zfill(j, _):
        pltpu.sync_copy(r0, acc.at[pl.ds(rs + j * _ZR, _ZR), :])
        return 0

    lax.fori_loop(0, _NZ, zfill, 0)
    pltpu.sync_copy(r0.at[pl.ds(0, _ZT), :],
                    acc.at[pl.ds(rs + _NZ * _ZR, _ZT), :])
    plsc.subcore_barrier()

    gc = gidx_hbm.at[cid]
    gbufs = (gidx_v0, gidx_v1)
    dbufs = (dst_v0, dst_v1)
    ebufs = (en_v0, en_v1)
    rbufs = (rows_v0, rows_v1)
    sems = (sem0, sem1)

    def _issue(g, b):
        off = pl.multiple_of(sid * EPT + g * KC, 8)
        pltpu.sync_copy(gc.at[pl.ds(off, KC)], gbufs[b])
        pltpu.sync_copy(dst_hbm.at[pl.ds(off, KC)], dbufs[b])
        pltpu.sync_copy(en_hbm.at[pl.ds(off, KC)], ebufs[b])
        pltpu.async_copy(tbl_hbm.at[gbufs[b]], rbufs[b], sems[b])

    _issue(0, 0)
    _issue(1, 1)

    def _drain(b):
        pltpu.make_async_copy(tbl_hbm.at[gbufs[b]], rbufs[b],
                              sems[b]).wait()
        rb = rbufs[b]
        eb = ebufs[b]

        @plsc.parallel_loop(0, KC, unroll=8)
        def _scale(i):
            e = plsc.load_gather(eb, [jnp.full((16,), i, jnp.int32)])
            rb[i, :] = rb[i, :] * e

        pltpu.sync_copy(rb, acc.at[dbufs[b]], add=True)

    def chunk2(j, _):
        for b in range(2):
            g = 2 * j + b
            _drain(b)

            @pl.when(g + 2 < NCH)
            def _():
                _issue(g + 2, b)
        return 0

    lax.fori_loop(0, NPAIR, chunk2, 0)
    _drain(0)
    plsc.subcore_barrier()

    def flush(j, _):
        pltpu.sync_copy(acc.at[pl.ds(rs + j * _ZR, _ZR), :], r0)
        pltpu.sync_copy(r0, agg_hbm.at[cid, pl.ds(rs + j * _ZR, _ZR), :])
        return 0

    lax.fori_loop(0, _NZ, flush, 0)
    tail = rs + _NZ * _ZR
    pltpu.sync_copy(acc.at[pl.ds(tail, _ZT), :], r0.at[pl.ds(0, _ZT), :])
    pltpu.sync_copy(r0.at[pl.ds(0, _ZT), :],
                    agg_hbm.at[cid, pl.ds(tail, _ZT), :])


def _conv_agg(tbl, gidx, dst, enorm):
    return pl.kernel(
        _conv_body,
        out_type=jax.ShapeDtypeStruct((NC, NP, HF), jnp.float32),
        mesh=_MESH,
        scratch_types=[
            pltpu.VMEM_SHARED((NP, HF), jnp.float32),
            pltpu.VMEM((KC,), jnp.int32),
            pltpu.VMEM((KC,), jnp.int32),
            pltpu.VMEM((KC,), jnp.int32),
            pltpu.VMEM((KC,), jnp.int32),
            pltpu.VMEM((KC,), jnp.float32),
            pltpu.VMEM((KC,), jnp.float32),
            pltpu.VMEM((KC, HF), jnp.float32),
            pltpu.VMEM((KC, HF), jnp.float32),
            pltpu.SemaphoreType.DMA,
            pltpu.SemaphoreType.DMA,
        ],
        compiler_params=pltpu.CompilerParams(
            use_tc_tiling_on_sc=False, needs_layout_passes=False
        ),
    )(tbl, gidx, dst, enorm)


_BW = 8000


def _packw1_body(w_ref, o_ref):
    x = w_ref[...]
    xr = x.reshape(_BW // 4, 4, H)
    for k in range(4):
        o_ref[:, k * H:(k + 1) * H] = xr[:, k, :]


def _packw1(w1flat):
    return pl.pallas_call(
        _packw1_body,
        grid=(NR // _BW,),
        in_specs=[pl.BlockSpec((_BW, H), lambda i: (i, 0))],
        out_specs=pl.BlockSpec((_BW // 4, 128), lambda i: (i, 0)),
        out_shape=jax.ShapeDtypeStruct((NR // 4, 128), jnp.float32),
    )(w1flat)


def _w2cat_body(w2_ref, o_ref):
    w2 = w2_ref[...]
    for c in range(NC):
        o_ref[c, :, :] = jnp.concatenate(
            [w2[r][:, c * HF:(c + 1) * HF] for r in range(R)], axis=1)


def _w2cat(w2):
    return pl.pallas_call(
        _w2cat_body,
        out_shape=jax.ShapeDtypeStruct((NC, H, R * HF), jnp.float32),
    )(w2)


_BN = 1000


def _dense1_body(agg_ref, root1_ref, b1_ref, w2c_ref, root2_ref, b2_ref,
                 xw_ref, z_ref):
    a = agg_ref[...]
    h = jnp.concatenate([a[0], a[1]], axis=1) + root1_ref[...] + b1_ref[...]
    h = jnp.maximum(h, 0.0)
    w2c = w2c_ref[...]
    for c in range(NC):
        xw_ref[c, :, :] = jnp.dot(h, w2c[c],
                                  preferred_element_type=jnp.float32)
    z_ref[...] = (jnp.dot(h, root2_ref[...], preferred_element_type=jnp.float32)
                  + b2_ref[...])


def _dense1(agg1, root1, bias1, w2c, root2, bias2):
    grid = (N // _BN,)
    xw, z = pl.pallas_call(
        _dense1_body,
        grid=grid,
        in_specs=[
            pl.BlockSpec((NC, _BN, HF), lambda i: (0, i, 0)),
            pl.BlockSpec((_BN, H), lambda i: (i, 0)),
            pl.BlockSpec((1, H), lambda i: (0, 0)),
            pl.BlockSpec((NC, H, R * HF), lambda i: (0, 0, 0)),
            pl.BlockSpec((H, H), lambda i: (0, 0)),
            pl.BlockSpec((1, H), lambda i: (0, 0)),
        ],
        out_specs=[
            pl.BlockSpec((NC, _BN, R * HF), lambda i: (0, i, 0)),
            pl.BlockSpec((_BN, H), lambda i: (i, 0)),
        ],
        out_shape=[
            jax.ShapeDtypeStruct((NC, N, R * HF), jnp.float32),
            jax.ShapeDtypeStruct((N, H), jnp.float32),
        ],
    )(agg1, root1, bias1.reshape(1, H), w2c, root2, bias2.reshape(1, H))
    return xw, z


def _final_body(agg_ref, z_ref, lw_ref, lb_ref, o_ref):
    a = agg_ref[...]
    h2 = jnp.concatenate([a[0], a[1]], axis=1) + z_ref[...]
    h2 = jnp.maximum(h2, 0.0)
    o_ref[...] = (jnp.dot(h2, lw_ref[...], preferred_element_type=jnp.float32)
                  + lb_ref[...])


def _final(agg2, z, lin_w, lin_b):
    grid = (N // _BN,)
    return pl.pallas_call(
        _final_body,
        grid=grid,
        in_specs=[
            pl.BlockSpec((NC, _BN, HF), lambda i: (0, i, 0)),
            pl.BlockSpec((_BN, H), lambda i: (i, 0)),
            pl.BlockSpec((H, C), lambda i: (0, 0)),
            pl.BlockSpec((1, C), lambda i: (0, 0)),
        ],
        out_specs=pl.BlockSpec((_BN, C), lambda i: (i, 0)),
        out_shape=jax.ShapeDtypeStruct((N, C), jnp.float32),
    )(agg2, z, lin_w, lin_b.reshape(1, C))


def kernel(weight1, root1, bias1, weight2, root2, bias2, lin_w, lin_b,
           edge_index, edge_type):
    src = edge_index[0]
    dst = edge_index[1]
    gidx1, gidx2, key = _prep(src, dst, edge_type)

    enorm = _hist_norm(key)

    w1rows = _packw1(weight1.reshape(NR, H)).reshape(NR * NC, HF)
    agg1 = _conv_agg(w1rows, gidx1, dst, enorm)

    w2c = _w2cat(weight2)
    xw, z = _dense1(agg1, root1, bias1, w2c, root2, bias2)

    xwrows = xw.reshape(NC * N * R, HF)
    agg2 = _conv_agg(xwrows, gidx2, dst, enorm)
    return _final(agg2, z, lin_w, lin_b)
